# baseline (speedup 1.0000x reference)
import jax
import jax.numpy as jnp
from jax.experimental import pallas as pl
from jax.experimental.pallas import tpu as pltpu

_THRESH = 0.5
_NCHUNK = 8
_G = 16 // _NCHUNK


def _gate_body(ent_hbm, out_hbm, ent_v, out_v, in_sem, out_sem):
    for k in range(_NCHUNK):
        pltpu.make_async_copy(
            ent_hbm.at[pl.ds(_G * k, _G)], ent_v.at[k], in_sem.at[k]).start()
    for k in range(_NCHUNK):
        pltpu.make_async_copy(
            ent_hbm.at[pl.ds(_G * k, _G)], ent_v.at[k], in_sem.at[k]).wait()
        m = jnp.where(ent_v[k] > _THRESH, jnp.int32(65536), jnp.int32(1))
        out_v[k] = pltpu.bitcast(m, jnp.int16).astype(jnp.int32)
        pltpu.make_async_copy(
            out_v.at[k], out_hbm.at[pl.ds(_G * k, _G)], out_sem.at[k]).start()
    for k in range(_NCHUNK):
        pltpu.make_async_copy(
            out_v.at[k], out_hbm.at[pl.ds(_G * k, _G)], out_sem.at[k]).wait()


@jax.jit
def kernel(h_fine, h_coarse, entropy):
    del h_fine, h_coarse
    rows = pl.pallas_call(
        _gate_body,
        in_specs=[pl.BlockSpec(memory_space=pltpu.MemorySpace.HBM)],
        out_specs=pl.BlockSpec(memory_space=pltpu.MemorySpace.HBM),
        scratch_shapes=[
            pltpu.VMEM((_NCHUNK, _G, 32, 32), jnp.float32),
            pltpu.VMEM((_NCHUNK, _G, 64, 32), jnp.int32),
            pltpu.SemaphoreType.DMA((_NCHUNK,)),
            pltpu.SemaphoreType.DMA((_NCHUNK,)),
        ],
        out_shape=jax.ShapeDtypeStruct((16, 64, 32), jnp.int32),
    )(entropy)
    return rows.reshape(16, 32, 2, 32).transpose(0, 1, 3, 2)


# speedup vs baseline: 1.0378x; 1.0378x over previous
import jax
import jax.numpy as jnp
from jax.experimental import pallas as pl
from jax.experimental.pallas import tpu as pltpu

_THRESH = 0.5
_NCHUNK = 2
_G = 16 // _NCHUNK


def _gate_body(ent_hbm, out_hbm, ent_v, out_v, in_sem, out_sem):
    for k in range(_NCHUNK):
        pltpu.make_async_copy(
            ent_hbm.at[pl.ds(_G * k, _G)], ent_v.at[k], in_sem.at[k]).start()
    for k in range(_NCHUNK):
        pltpu.make_async_copy(
            ent_hbm.at[pl.ds(_G * k, _G)], ent_v.at[k], in_sem.at[k]).wait()
        m = jnp.where(ent_v[k] > _THRESH, jnp.int32(65536), jnp.int32(1))
        out_v[k] = pltpu.bitcast(m, jnp.int16).astype(jnp.int32)
        pltpu.make_async_copy(
            out_v.at[k], out_hbm.at[pl.ds(_G * k, _G)], out_sem.at[k]).start()
    for k in range(_NCHUNK):
        pltpu.make_async_copy(
            out_v.at[k], out_hbm.at[pl.ds(_G * k, _G)], out_sem.at[k]).wait()


@jax.jit
def kernel(h_fine, h_coarse, entropy):
    del h_fine, h_coarse
    rows = pl.pallas_call(
        _gate_body,
        in_specs=[pl.BlockSpec(memory_space=pltpu.MemorySpace.HBM)],
        out_specs=pl.BlockSpec(memory_space=pltpu.MemorySpace.HBM),
        scratch_shapes=[
            pltpu.VMEM((_NCHUNK, _G, 32, 32), jnp.float32),
            pltpu.VMEM((_NCHUNK, _G, 64, 32), jnp.int32),
            pltpu.SemaphoreType.DMA((_NCHUNK,)),
            pltpu.SemaphoreType.DMA((_NCHUNK,)),
        ],
        out_shape=jax.ShapeDtypeStruct((16, 64, 32), jnp.int32),
    )(entropy)
    return rows.reshape(16, 32, 2, 32).transpose(0, 1, 3, 2)
